# baseline (speedup 1.0000x reference)
import jax
import jax.numpy as jnp
from jax.experimental import pallas as pl

_B = 64
_T = 16
_S = 10
_D = 128
_G = 1000
_GP = 1024
_ST = 64
_TOT = 4096
_R = _B * _S

_HI = jax.lax.Precision.HIGHEST


def _dot(a, b):
    return jax.lax.dot_general(a, b, (((a.ndim - 1,), (0,)), ((), ())),
                               precision=_HI, preferred_element_type=jnp.float32)


def _cumsum_lanes(x):
    r, n = x.shape
    d = 1
    while d < n:
        x = x + jnp.concatenate(
            [jnp.zeros((r, d), x.dtype), x[:, :-d]], axis=1)
        d *= 2
    return x


def _body(zi_ref, latent_ref, rep_ref, wp_ref, bp_ref, em_ref, rd_ref,
          wv_ref, bv_ref, bvc_ref, emt_ref, wvr_ref, y_ref, tri_ref, out_ref):
    g = _dot(em_ref[...], wv_ref[...]) + bv_ref[...]
    gm = jnp.max(g, axis=-1, keepdims=True)
    lse_state = gm + jnp.log(jnp.sum(jnp.exp(g - gm), axis=-1, keepdims=True))
    gt = _dot(wvr_ref[...], emt_ref[...]) + bvc_ref[...]
    iota_gp = jax.lax.broadcasted_iota(jnp.int32, (_T * _B, _GP), 1)
    onehot_y = (y_ref[...] == iota_gp).astype(jnp.float32)
    h = _dot(onehot_y, gt)
    m_all = h - lse_state.reshape(1, _ST)

    iota_tot = jax.lax.broadcasted_iota(jnp.int32, (_B, _TOT), 1)
    onehot_zi = (zi_ref[...] == iota_tot).astype(jnp.float32)
    z0 = _dot(onehot_zi, latent_ref[...])
    z = _dot(rep_ref[...], z0)
    zold = jnp.zeros_like(z)
    iota_st = jax.lax.broadcasted_iota(jnp.int32, (_R, _ST), 1)
    ones_d = jnp.ones((_D, 1), jnp.float32)
    ones_st = jnp.ones((_ST, 1), jnp.float32)
    for i in range(_T):
        mean = _dot(z, ones_d) * (1.0 / _D)
        c = z - mean
        var = _dot(c * c, ones_d) * (1.0 / (_D - 1))
        z = z / (1e-05 + jnp.sqrt(var)) * 0.113
        pref = _dot(jnp.concatenate([zold, z], axis=-1), wp_ref[...]) + bp_ref[...]
        e = jnp.exp(pref)
        xp = e / _dot(e, ones_st)
        xpc = _dot(xp, tri_ref[...])
        rd = rd_ref[i]
        cnt = _dot((xpc <= rd).astype(jnp.float32), ones_st).astype(jnp.int32)
        which = jnp.where(cnt >= _ST, 0, cnt)
        onehot = (which == iota_st).astype(jnp.float32)
        mvs = _dot(onehot, em_ref[...])

        mt_exp = _dot(rep_ref[...], m_all[i * _B:(i + 1) * _B, :])
        yp = jnp.sum(onehot * mt_exp, axis=-1, keepdims=True)
        grp = jax.lax.dot_general(yp, rep_ref[...], (((0,), (0,)), ((), ())),
                                  precision=_HI,
                                  preferred_element_type=jnp.float32)
        out_ref[i] = grp * (1.0 / _S)

        zold = z
        z = z + mvs


def kernel(zi, y, latent, W_v, b_v, W_p, b_p, E_m):
    zi2 = zi.astype(jnp.int32).reshape(_B, 1)
    rep = (jnp.arange(_R)[:, None] // _S == jnp.arange(_B)[None, :]
           ).astype(jnp.float32)
    wp = W_p.T
    bp = b_p.reshape(1, _ST)
    rkey = jax.random.key(42)
    rd = jnp.stack([
        jax.random.uniform(jax.random.fold_in(rkey, i), (_B, _S),
                           dtype=jnp.float32).reshape(_R)
        for i in range(_T)
    ]).reshape(_T, _R, 1)
    wv = jnp.zeros((_D, _GP), jnp.float32).at[:, :_G].set(W_v.T)
    bv = jnp.full((1, _GP), -1e30, jnp.float32).at[0, :_G].set(b_v)
    bvc = bv.reshape(_GP, 1)
    emt = E_m.T
    wvr = jnp.zeros((_GP, _D), jnp.float32).at[:_G, :].set(W_v)
    y_t = y.astype(jnp.int32).T.reshape(_T * _B, 1)
    tri = (jnp.arange(_ST)[:, None] <= jnp.arange(_ST)[None, :]
           ).astype(jnp.float32)

    out_t = pl.pallas_call(
        _body,
        out_shape=jax.ShapeDtypeStruct((_T, 1, _B), jnp.float32),
    )(zi2, latent, rep, wp, bp, E_m, rd, wv, bv, bvc, emt, wvr, y_t, tri)

    return out_t.reshape(_T, _B).T


# speedup vs baseline: 1.2199x; 1.2199x over previous
import jax
import jax.numpy as jnp
from jax.experimental import pallas as pl

_B = 64
_T = 16
_S = 10
_D = 128
_G = 1000
_GP = 1024
_ST = 64
_TOT = 4096
_R = _B * _S

_HI = jax.lax.Precision.HIGHEST


def _dot(a, b):
    return jax.lax.dot_general(a, b, (((a.ndim - 1,), (0,)), ((), ())),
                               precision=_HI, preferred_element_type=jnp.float32)


def _cumsum_lanes(x):
    r, n = x.shape
    d = 1
    while d < n:
        x = x + jnp.concatenate(
            [jnp.zeros((r, d), x.dtype), x[:, :-d]], axis=1)
        d *= 2
    return x


def _body(zi_ref, latent_ref, rep_ref, rept_ref, wp_ref, bp_ref, em_ref,
          rd_ref, wv_ref, bv_ref, bvc_ref, emt_ref, wvr_ref, y_ref, tri_ref,
          out_ref):
    g = _dot(em_ref[...], wv_ref[...]) + bv_ref[...]
    gm = jnp.max(g, axis=-1, keepdims=True)
    lse_state = gm + jnp.log(jnp.sum(jnp.exp(g - gm), axis=-1, keepdims=True))
    gt = _dot(wvr_ref[...], emt_ref[...]) + bvc_ref[...]
    iota_gp = jax.lax.broadcasted_iota(jnp.int32, (_T * _B, _GP), 1)
    onehot_y = (y_ref[...] == iota_gp).astype(jnp.float32)
    h = _dot(onehot_y, gt)
    m_all = h - lse_state.reshape(1, _ST)

    iota_tot = jax.lax.broadcasted_iota(jnp.int32, (_B, _TOT), 1)
    onehot_zi = (zi_ref[...] == iota_tot).astype(jnp.float32)
    z0 = _dot(onehot_zi, latent_ref[...])
    z = _dot(rep_ref[...], z0)
    zold = jnp.zeros_like(z)
    iota_st = jax.lax.broadcasted_iota(jnp.int32, (_R, _ST), 1)
    s_list = []
    for i in range(_T):
        mean = jnp.mean(z, axis=-1, keepdims=True)
        c = z - mean
        var = jnp.sum(c * c, axis=-1, keepdims=True) * (1.0 / (_D - 1))
        z = z / (1e-05 + jnp.sqrt(var)) * 0.113
        pref = _dot(jnp.concatenate([zold, z], axis=-1), wp_ref[...]) + bp_ref[...]
        m = jnp.max(pref, axis=-1, keepdims=True)
        e = jnp.exp(pref - m)
        xp = e / jnp.sum(e, axis=-1, keepdims=True)
        xpc = _dot(xp, tri_ref[...])
        rd = rd_ref[i]
        cnt = jnp.sum((xpc <= rd).astype(jnp.int32), axis=-1, keepdims=True)
        which = jnp.where(cnt >= _ST, 0, cnt)
        onehot = (which == iota_st).astype(jnp.float32)
        mvs = _dot(onehot, em_ref[...])
        s_list.append(_dot(rept_ref[...], onehot))

        zold = z
        z = z + mvs

    s_all = jnp.concatenate(s_list, axis=0)
    out_ref[...] = jnp.sum(s_all * m_all, axis=-1, keepdims=True) * (1.0 / _S)


def kernel(zi, y, latent, W_v, b_v, W_p, b_p, E_m):
    zi2 = zi.astype(jnp.int32).reshape(_B, 1)
    rep = (jnp.arange(_R)[:, None] // _S == jnp.arange(_B)[None, :]
           ).astype(jnp.float32)
    wp = W_p.T
    bp = b_p.reshape(1, _ST)
    rkey = jax.random.key(42)
    rd = jnp.stack([
        jax.random.uniform(jax.random.fold_in(rkey, i), (_B, _S),
                           dtype=jnp.float32).reshape(_R)
        for i in range(_T)
    ]).reshape(_T, _R, 1)
    wv = jnp.zeros((_D, _GP), jnp.float32).at[:, :_G].set(W_v.T)
    bv = jnp.full((1, _GP), -1e30, jnp.float32).at[0, :_G].set(b_v)
    bvc = bv.reshape(_GP, 1)
    emt = E_m.T
    wvr = jnp.zeros((_GP, _D), jnp.float32).at[:_G, :].set(W_v)
    y_t = y.astype(jnp.int32).T.reshape(_T * _B, 1)
    tri = (jnp.arange(_ST)[:, None] <= jnp.arange(_ST)[None, :]
           ).astype(jnp.float32)

    out_t = pl.pallas_call(
        _body,
        out_shape=jax.ShapeDtypeStruct((_T * _B, 1), jnp.float32),
    )(zi2, latent, rep, rep.T, wp, bp, E_m, rd, wv, bv, bvc, emt, wvr, y_t,
      tri)

    return out_t.reshape(_T, _B).T


# speedup vs baseline: 1.2586x; 1.0318x over previous
import jax
import jax.numpy as jnp
from jax.experimental import pallas as pl

_B = 64
_T = 16
_S = 10
_D = 128
_G = 1000
_GP = 1024
_ST = 64
_TOT = 4096
_R = _B * _S

_HI = jax.lax.Precision.HIGHEST


def _dot(a, b):
    return jax.lax.dot_general(a, b, (((a.ndim - 1,), (0,)), ((), ())),
                               precision=_HI, preferred_element_type=jnp.float32)


def _body(zi_ref, latent_ref, rep_ref, wp_ref, bp_ref, em_ref,
          rd_ref, wv_ref, bv_ref, bvc_ref, emt_ref, wvr_ref, y_ref, tri_ref,
          out_ref):
    g = _dot(em_ref[...], wv_ref[...]) + bv_ref[...]
    gm = jnp.max(g, axis=-1, keepdims=True)
    lse_state = gm + jnp.log(jnp.sum(jnp.exp(g - gm), axis=-1, keepdims=True))
    gt = _dot(wvr_ref[...], emt_ref[...]) + bvc_ref[...]
    iota_gp = jax.lax.broadcasted_iota(jnp.int32, (_T * _B, _GP), 1)
    onehot_y = (y_ref[...] == iota_gp).astype(jnp.float32)
    h = _dot(onehot_y, gt)
    m_all = h - lse_state.reshape(1, _ST)

    iota_tot = jax.lax.broadcasted_iota(jnp.int32, (_B, _TOT), 1)
    onehot_zi = (zi_ref[...] == iota_tot).astype(jnp.float32)
    z0 = _dot(onehot_zi, latent_ref[...])
    z = _dot(rep_ref[...], z0)
    zold = jnp.zeros_like(z)
    iota_st = jax.lax.broadcasted_iota(jnp.int32, (_R, _ST), 1)
    for i in range(_T):
        mean = jnp.mean(z, axis=-1, keepdims=True)
        c = z - mean
        var = jnp.sum(c * c, axis=-1, keepdims=True) * (1.0 / (_D - 1))
        z = z / (1e-05 + jnp.sqrt(var)) * 0.113
        pref = _dot(jnp.concatenate([zold, z], axis=-1), wp_ref[...]) + bp_ref[...]
        m = jnp.max(pref, axis=-1, keepdims=True)
        e = jnp.exp(pref - m)
        xp = e / jnp.sum(e, axis=-1, keepdims=True)
        xpc = _dot(xp, tri_ref[...])
        rd = rd_ref[i]
        cnt = jnp.sum((xpc <= rd).astype(jnp.int32), axis=-1, keepdims=True)
        which = jnp.where(cnt >= _ST, 0, cnt)
        onehot = (which == iota_st).astype(jnp.float32)
        mvs = _dot(onehot, em_ref[...])

        mt_exp = _dot(rep_ref[...], m_all[i * _B:(i + 1) * _B, :])
        yp = jnp.sum(onehot * mt_exp, axis=-1, keepdims=True)
        grp = jax.lax.dot_general(yp, rep_ref[...], (((0,), (0,)), ((), ())),
                                  precision=_HI,
                                  preferred_element_type=jnp.float32)
        out_ref[i] = grp * (1.0 / _S)

        zold = z
        z = z + mvs


def kernel(zi, y, latent, W_v, b_v, W_p, b_p, E_m):
    zi2 = zi.astype(jnp.int32).reshape(_B, 1)
    rep = (jnp.arange(_R)[:, None] // _S == jnp.arange(_B)[None, :]
           ).astype(jnp.float32)
    wp = W_p.T
    bp = b_p.reshape(1, _ST)
    rkey = jax.random.key(42)
    rd = jnp.stack([
        jax.random.uniform(jax.random.fold_in(rkey, i), (_B, _S),
                           dtype=jnp.float32).reshape(_R)
        for i in range(_T)
    ]).reshape(_T, _R, 1)
    wv = jnp.zeros((_D, _GP), jnp.float32).at[:, :_G].set(W_v.T)
    bv = jnp.full((1, _GP), -1e30, jnp.float32).at[0, :_G].set(b_v)
    bvc = bv.reshape(_GP, 1)
    emt = E_m.T
    wvr = jnp.zeros((_GP, _D), jnp.float32).at[:_G, :].set(W_v)
    y_t = y.astype(jnp.int32).T.reshape(_T * _B, 1)
    tri = (jnp.arange(_ST)[:, None] <= jnp.arange(_ST)[None, :]
           ).astype(jnp.float32)

    out_t = pl.pallas_call(
        _body,
        out_shape=jax.ShapeDtypeStruct((_T, 1, _B), jnp.float32),
    )(zi2, latent, rep, wp, bp, E_m, rd, wv, bv, bvc, emt, wvr, y_t, tri)

    return out_t.reshape(_T, _B).T
